# baseline (speedup 1.0000x reference)
import functools

import jax
import jax.numpy as jnp
from jax import lax
from jax.experimental import pallas as pl
from jax.experimental.pallas import tpu as pltpu
from jax.experimental.pallas import tpu_sc as plsc

N = 10000
C = 128
E = 320000
NC, NS = 2, 16
NW = NC * NS
K = 128
CH = 80
EPW = CH * K
EP = NW * EPW
ACC_R = 10240
DUMMY = N
HL = 10240
RPT = ACC_R // NS
HC = C // 2
ECH = 161
EPT = ECH * K
EP16 = NS * EPT

def _make_sc_kernels():
    mesh = plsc.VectorSubcoreMesh(
        core_axis_name="c", subcore_axis_name="s",
        num_cores=NC, num_subcores=NS)
    sc_hist = functools.partial(
        pl.kernel,
        out_type=jax.ShapeDtypeStruct((NC, ACC_R, HC), jnp.float32),
        mesh=mesh,
        scratch_types=[
            pltpu.VMEM((CH, K), jnp.int32),
            pltpu.VMEM((K, HC), jnp.float32),
            pltpu.VMEM((K, HC), jnp.float32),
            pltpu.VMEM_SHARED((ACC_R, HC), jnp.float32),
        ],
        compiler_params=pltpu.CompilerParams(use_tc_tiling_on_sc=False),
    )(_sc_hist_body)
    sc_scatter = functools.partial(
        pl.kernel,
        out_type=jax.ShapeDtypeStruct((NC, ACC_R, HC), jnp.float32),
        mesh=mesh,
        scratch_types=[
            pltpu.VMEM((ECH, K), jnp.int32),
            pltpu.VMEM((ECH, K), jnp.int32),
            pltpu.VMEM((K, HC), jnp.float32),
            pltpu.VMEM((K, HC), jnp.float32),
            pltpu.SemaphoreType.DMA,
            pltpu.SemaphoreType.DMA,
            pltpu.VMEM_SHARED((ACC_R, HC), jnp.float32),
        ],
        compiler_params=pltpu.CompilerParams(use_tc_tiling_on_sc=False),
    )(_sc_scatter_body)
    return sc_hist, sc_scatter


def _sc_hist_body(col_hbm, hist_hbm, col_v, ones_v, zbuf, shared):
    cid = lax.axis_index("c")
    sid = lax.axis_index("s")
    wid = sid * NC + cid
    pltpu.sync_copy(col_hbm.at[wid], col_v)
    zero16 = jnp.zeros((16,), jnp.float32)
    ones16 = jnp.ones((16,), jnp.float32)

    @pl.loop(0, K * HC // 16)
    def _init(i):
        r = i // (HC // 16)
        s = pl.ds((i % (HC // 16)) * 16, 16)
        ones_v[r, s] = ones16
        zbuf[r, s] = zero16

    base = sid * RPT
    for r in range(RPT // K):
        pltpu.sync_copy(zbuf, shared.at[pl.ds(base + r * K, K)])
    plsc.subcore_barrier()

    @pl.loop(0, CH)
    def _acc(j):
        pltpu.sync_copy(ones_v, shared.at[col_v.at[j]], add=True)

    plsc.subcore_barrier()
    pltpu.sync_copy(shared.at[pl.ds(base, RPT)],
                    hist_hbm.at[cid, pl.ds(base, RPT)])


def _sc_scatter_body(h2_hbm, row_hbm, col_hbm, out_hbm,
                     row_v, col_v, b0, b1, gs0, gs1, acc):
    cid = lax.axis_index("c")
    sid = lax.axis_index("s")
    pltpu.sync_copy(row_hbm.at[cid, sid], row_v)
    pltpu.sync_copy(col_hbm.at[sid], col_v)

    zero16 = jnp.zeros((16,), jnp.float32)

    @pl.loop(0, K * HC // 16)
    def _z(k):
        b0[k // (HC // 16), pl.ds((k % (HC // 16)) * 16, 16)] = zero16

    base = sid * RPT
    for r in range(RPT // K):
        pltpu.sync_copy(b0, acc.at[pl.ds(base + r * K, K)])
    plsc.subcore_barrier()

    pltpu.async_copy(h2_hbm.at[row_v.at[0]], b0, gs0).wait()

    @pl.loop(0, ECH // 2)
    def _body(p):
        j = p * 2
        cpa = pltpu.async_copy(h2_hbm.at[row_v.at[j + 1]], b1, gs1)
        pltpu.sync_copy(b0, acc.at[col_v.at[j]], add=True)
        cpa.wait()
        pltpu.async_copy(h2_hbm.at[row_v.at[j + 2]], b0, gs0)
        pltpu.sync_copy(b1, acc.at[col_v.at[j + 1]], add=True)
        pltpu.make_async_copy(h2_hbm.at[row_v.at[0]], b0, gs0).wait()

    pltpu.sync_copy(b0, acc.at[col_v.at[ECH - 1]], add=True)

    plsc.subcore_barrier()
    pltpu.sync_copy(acc.at[pl.ds(base, RPT)],
                    out_hbm.at[cid, pl.ds(base, RPT)])


def _tc_h2_body(x_ref, w_ref, hist_ref, h2_ref):
    deg = hist_ref[0] + hist_ref[1] + 1.0
    dis = lax.rsqrt(deg)
    h = jnp.dot(x_ref[...], w_ref[...], preferred_element_type=jnp.float32)
    h2_ref[...] = h * dis


def _tc_fin_body(acc_ref, h2_ref, hist_ref, b_ref, out_ref):
    deg = hist_ref[0] + hist_ref[1] + 1.0
    dis = lax.rsqrt(deg)
    agg = jnp.concatenate([acc_ref[0, :N], acc_ref[1, :N]], axis=1)
    s = (agg + h2_ref[...]) * dis + b_ref[...]
    out_ref[...] = jnp.maximum(s, 0.0)


def kernel(x, edge_index, W, b):
    ei = edge_index.astype(jnp.int32)
    row, col = ei[0], ei[1]
    pad = EP - E
    col3 = jnp.concatenate(
        [col, jnp.full((pad,), DUMMY, jnp.int32)]).reshape(NW, CH, K)
    pad16 = EP16 - E
    row_p = jnp.concatenate([row, jnp.zeros((pad16,), jnp.int32)])
    col_p = jnp.concatenate([col, jnp.full((pad16,), DUMMY, jnp.int32)])
    rowx = jnp.stack([2 * row_p, 2 * row_p + 1]).reshape(NC, NS, ECH, K)
    col16 = col_p.reshape(NS, ECH, K)

    sc_hist, sc_scatter = _make_sc_kernels()
    hist = sc_hist(col3)
    hist_n = hist[:, :N, :1]

    h2 = pl.pallas_call(
        _tc_h2_body,
        out_shape=jax.ShapeDtypeStruct((N, C), jnp.float32),
    )(x, W, hist_n)
    h2v = h2.reshape(2 * N, HC)

    accs = sc_scatter(h2v, rowx, col16)

    out = pl.pallas_call(
        _tc_fin_body,
        out_shape=jax.ShapeDtypeStruct((N, C), jnp.float32),
    )(accs, h2, hist_n, b)
    return out


# speedup vs baseline: 2.0830x; 2.0830x over previous
import functools

import jax
import jax.numpy as jnp
from jax import lax
from jax.experimental import pallas as pl
from jax.experimental.pallas import tpu as pltpu
from jax.experimental.pallas import tpu_sc as plsc

N = 10000
C = 128
E = 320000
NC, NS = 2, 16
NW = NC * NS
K = 128
CH = 80
EPW = CH * K
EP = NW * EPW
ACC_R = 10240
DUMMY = N
HL = 10240
RPT = ACC_R // NS
HC = C // 2
ECH = 161
EPT = ECH * K
EP16 = NS * EPT

def _make_sc_kernels():
    mesh = plsc.VectorSubcoreMesh(
        core_axis_name="c", subcore_axis_name="s",
        num_cores=NC, num_subcores=NS)
    sc_hist = functools.partial(
        pl.kernel,
        out_type=jax.ShapeDtypeStruct((NC, ACC_R, HC), jnp.float32),
        mesh=mesh,
        scratch_types=[
            pltpu.VMEM((CH, K), jnp.int32),
            pltpu.VMEM((K, HC), jnp.float32),
            pltpu.VMEM((K, HC), jnp.float32),
            pltpu.VMEM_SHARED((ACC_R, HC), jnp.float32),
        ],
        compiler_params=pltpu.CompilerParams(use_tc_tiling_on_sc=False),
    )(_sc_hist_body)
    sc_scatter = functools.partial(
        pl.kernel,
        out_type=jax.ShapeDtypeStruct((NC, ACC_R, HC), jnp.float32),
        mesh=mesh,
        scratch_types=[
            pltpu.VMEM((ECH, K), jnp.int32),
            pltpu.VMEM((ECH, K), jnp.int32),
            pltpu.VMEM((K, HC), jnp.float32),
            pltpu.VMEM((K, HC), jnp.float32),
            pltpu.SemaphoreType.DMA,
            pltpu.SemaphoreType.DMA,
            pltpu.VMEM_SHARED((ACC_R, HC), jnp.float32),
        ],
        compiler_params=pltpu.CompilerParams(use_tc_tiling_on_sc=False),
    )(_sc_scatter_body)
    return sc_hist, sc_scatter


def _sc_hist_body(col_hbm, hist_hbm, col_v, ones_v, zbuf, shared):
    cid = lax.axis_index("c")
    sid = lax.axis_index("s")
    wid = sid * NC + cid
    pltpu.sync_copy(col_hbm.at[wid], col_v)
    zero16 = jnp.zeros((16,), jnp.float32)
    ones16 = jnp.ones((16,), jnp.float32)

    @pl.loop(0, K * HC // 16)
    def _init(i):
        r = i // (HC // 16)
        s = pl.ds((i % (HC // 16)) * 16, 16)
        ones_v[r, s] = ones16
        zbuf[r, s] = zero16

    base = sid * RPT
    for r in range(RPT // K):
        pltpu.sync_copy(zbuf, shared.at[pl.ds(base + r * K, K)])
    plsc.subcore_barrier()

    @pl.loop(0, CH)
    def _acc(j):
        pltpu.sync_copy(ones_v, shared.at[col_v.at[j]], add=True)

    plsc.subcore_barrier()
    pltpu.sync_copy(shared.at[pl.ds(base, RPT)],
                    hist_hbm.at[cid, pl.ds(base, RPT)])


def _sc_scatter_body(h2_hbm, row_hbm, col_hbm, out_hbm,
                     row_v, col_v, b0, b1, gs0, gs1, acc):
    cid = lax.axis_index("c")
    sid = lax.axis_index("s")
    pltpu.sync_copy(row_hbm.at[cid, sid], row_v)
    pltpu.sync_copy(col_hbm.at[sid], col_v)

    zero16 = jnp.zeros((16,), jnp.float32)

    @pl.loop(0, K * HC // 16)
    def _z(k):
        b0[k // (HC // 16), pl.ds((k % (HC // 16)) * 16, 16)] = zero16

    base = sid * RPT
    for r in range(RPT // K):
        pltpu.sync_copy(b0, acc.at[pl.ds(base + r * K, K)])
    plsc.subcore_barrier()

    pltpu.async_copy(h2_hbm.at[row_v.at[0]], b0, gs0).wait()

    @pl.loop(0, ECH // 2)
    def _body(p):
        j = p * 2
        cpa = pltpu.async_copy(h2_hbm.at[row_v.at[j + 1]], b1, gs1)
        pltpu.sync_copy(b0, acc.at[col_v.at[j]], add=True)
        cpa.wait()
        pltpu.async_copy(h2_hbm.at[row_v.at[j + 2]], b0, gs0)
        pltpu.sync_copy(b1, acc.at[col_v.at[j + 1]], add=True)
        pltpu.make_async_copy(h2_hbm.at[row_v.at[0]], b0, gs0).wait()

    pltpu.sync_copy(b0, acc.at[col_v.at[ECH - 1]], add=True)

    plsc.subcore_barrier()
    pltpu.sync_copy(acc.at[pl.ds(base, RPT)],
                    out_hbm.at[cid, pl.ds(base, RPT)])


def _tc_h2_body(x_ref, w_ref, hist_ref, h2_ref):
    deg = hist_ref[0] + hist_ref[1] + 1.0
    dis = lax.rsqrt(deg)
    h = jnp.dot(x_ref[...], w_ref[...], preferred_element_type=jnp.float32)
    h2_ref[...] = h * dis


def _tc_fin_body(acc_ref, h2_ref, hist_ref, b_ref, out_ref):
    deg = hist_ref[0] + hist_ref[1] + 1.0
    dis = lax.rsqrt(deg)
    agg = jnp.concatenate([acc_ref[0, :N], acc_ref[1, :N]], axis=1)
    s = (agg + h2_ref[...]) * dis + b_ref[...]
    out_ref[...] = jnp.maximum(s, 0.0)


def kernel(x, edge_index, W, b):
    ei = edge_index.astype(jnp.int32)
    row, col = ei[0], ei[1]
    pad = EP - E
    col3 = jnp.concatenate(
        [col, jnp.full((pad,), DUMMY, jnp.int32)]).reshape(NW, CH, K)
    pad16 = EP16 - E
    pad_i = jnp.arange(pad16, dtype=jnp.int32)
    row_p = jnp.concatenate([row, pad_i % N])
    col_p = jnp.concatenate([col, DUMMY + pad_i % (ACC_R - N)])
    rowx = jnp.stack([2 * row_p, 2 * row_p + 1]).reshape(NC, NS, ECH, K)
    col16 = col_p.reshape(NS, ECH, K)

    sc_hist, sc_scatter = _make_sc_kernels()
    hist = sc_hist(col3)
    hist_n = hist[:, :N, :1]

    h2 = pl.pallas_call(
        _tc_h2_body,
        out_shape=jax.ShapeDtypeStruct((N, C), jnp.float32),
    )(x, W, hist_n)
    h2v = h2.reshape(2 * N, HC)

    accs = sc_scatter(h2v, rowx, col16)

    out = pl.pallas_call(
        _tc_fin_body,
        out_shape=jax.ShapeDtypeStruct((N, C), jnp.float32),
    )(accs, h2, hist_n, b)
    return out


# speedup vs baseline: 2.9608x; 1.4214x over previous
import functools

import jax
import jax.numpy as jnp
from jax import lax
from jax.experimental import pallas as pl
from jax.experimental.pallas import tpu as pltpu
from jax.experimental.pallas import tpu_sc as plsc

N = 10000
C = 128
E = 320000
NC, NS = 2, 16
NW = NC * NS
K = 128
CH = 80
EPW = CH * K
EP = NW * EPW
ACC_R = 10240
DUMMY = N
HL = 10240
RPT = ACC_R // NS
HC = C // 2
HW = 32
ECH = 162
EPT = ECH * K
EP16 = NS * EPT

def _make_sc_kernels():
    mesh = plsc.VectorSubcoreMesh(
        core_axis_name="c", subcore_axis_name="s",
        num_cores=NC, num_subcores=NS)
    sc_hist = functools.partial(
        pl.kernel,
        out_type=jax.ShapeDtypeStruct((NC, ACC_R, HW), jnp.float32),
        mesh=mesh,
        scratch_types=[
            pltpu.VMEM((CH, K), jnp.int32),
            pltpu.VMEM((K, HW), jnp.float32),
            pltpu.VMEM((K, HW), jnp.float32),
            pltpu.VMEM_SHARED((ACC_R, HW), jnp.float32),
        ],
        compiler_params=pltpu.CompilerParams(use_tc_tiling_on_sc=False),
    )(_sc_hist_body)
    sc_scatter = functools.partial(
        pl.kernel,
        out_type=jax.ShapeDtypeStruct((NC, ACC_R, HC), jnp.float32),
        mesh=mesh,
        scratch_types=[
            pltpu.VMEM((ECH, K), jnp.int32),
            pltpu.VMEM((ECH, K), jnp.int32),
            pltpu.VMEM((K, HC), jnp.float32),
            pltpu.VMEM((K, HC), jnp.float32),
            pltpu.VMEM((K, HC), jnp.float32),
            pltpu.SemaphoreType.DMA,
            pltpu.SemaphoreType.DMA,
            pltpu.SemaphoreType.DMA,
            pltpu.VMEM_SHARED((ACC_R, HC), jnp.float32),
        ],
        compiler_params=pltpu.CompilerParams(use_tc_tiling_on_sc=False),
    )(_sc_scatter_body)
    return sc_hist, sc_scatter


def _sc_hist_body(col_hbm, hist_hbm, col_v, ones_v, zbuf, shared):
    cid = lax.axis_index("c")
    sid = lax.axis_index("s")
    wid = sid * NC + cid
    pltpu.sync_copy(col_hbm.at[wid], col_v)
    zero16 = jnp.zeros((16,), jnp.float32)
    ones16 = jnp.ones((16,), jnp.float32)

    @pl.loop(0, K * HW // 16)
    def _init(i):
        r = i // (HW // 16)
        s = pl.ds((i % (HW // 16)) * 16, 16)
        ones_v[r, s] = ones16
        zbuf[r, s] = zero16

    base = sid * RPT
    for r in range(RPT // K):
        pltpu.sync_copy(zbuf, shared.at[pl.ds(base + r * K, K)])
    plsc.subcore_barrier()

    @pl.loop(0, CH)
    def _acc(j):
        pltpu.sync_copy(ones_v, shared.at[col_v.at[j]], add=True)

    plsc.subcore_barrier()
    pltpu.sync_copy(shared.at[pl.ds(base, RPT)],
                    hist_hbm.at[cid, pl.ds(base, RPT)])


def _sc_scatter_body(h2_hbm, row_hbm, col_hbm, out_hbm,
                     row_v, col_v, b0, b1, b2, gs0, gs1, gs2, acc):
    bufs = (b0, b1, b2)
    sems = (gs0, gs1, gs2)
    cid = lax.axis_index("c")
    sid = lax.axis_index("s")
    pltpu.sync_copy(row_hbm.at[cid, sid], row_v)
    pltpu.sync_copy(col_hbm.at[sid], col_v)

    zero16 = jnp.zeros((16,), jnp.float32)

    @pl.loop(0, K * HC // 16)
    def _z(k):
        b0[k // (HC // 16), pl.ds((k % (HC // 16)) * 16, 16)] = zero16

    base = sid * RPT
    for r in range(RPT // K):
        pltpu.sync_copy(b0, acc.at[pl.ds(base + r * K, K)])
    plsc.subcore_barrier()

    pltpu.async_copy(h2_hbm.at[row_v.at[0]], b0, gs0)
    pltpu.async_copy(h2_hbm.at[row_v.at[1]], b1, gs1)

    @pl.loop(0, (ECH - 3) // 3)
    def _body(t):
        j = t * 3
        for d in range(3):
            jj = j + d
            pltpu.make_async_copy(h2_hbm.at[row_v.at[jj]],
                                  bufs[d], sems[d]).wait()
            pltpu.async_copy(h2_hbm.at[row_v.at[jj + 2]],
                             bufs[(d + 2) % 3], sems[(d + 2) % 3])
            pltpu.sync_copy(bufs[d], acc.at[col_v.at[jj]], add=True)

    pltpu.async_copy(h2_hbm.at[row_v.at[ECH - 1]], b2, gs2)
    for d, (bb, ss) in enumerate(((b0, gs0), (b1, gs1), (b2, gs2))):
        jj = ECH - 3 + d
        pltpu.make_async_copy(h2_hbm.at[row_v.at[jj]], bb, ss).wait()
        pltpu.sync_copy(bb, acc.at[col_v.at[jj]], add=True)

    plsc.subcore_barrier()
    pltpu.sync_copy(acc.at[pl.ds(base, RPT)],
                    out_hbm.at[cid, pl.ds(base, RPT)])


def _tc_h2_body(x_ref, w_ref, hist_ref, h2_ref):
    deg = hist_ref[0] + hist_ref[1] + 1.0
    dis = lax.rsqrt(deg)
    h = jnp.dot(x_ref[...], w_ref[...], preferred_element_type=jnp.float32)
    h2_ref[...] = h * dis


def _tc_fin_body(acc_ref, h2_ref, hist_ref, b_ref, out_ref):
    deg = hist_ref[0] + hist_ref[1] + 1.0
    dis = lax.rsqrt(deg)
    agg = jnp.concatenate([acc_ref[0, :N], acc_ref[1, :N]], axis=1)
    s = (agg + h2_ref[...]) * dis + b_ref[...]
    out_ref[...] = jnp.maximum(s, 0.0)


def kernel(x, edge_index, W, b):
    ei = edge_index.astype(jnp.int32)
    row, col = ei[0], ei[1]
    pad = EP - E
    col3 = jnp.concatenate(
        [col, jnp.full((pad,), DUMMY, jnp.int32)]).reshape(NW, CH, K)
    pad16 = EP16 - E
    pad_i = jnp.arange(pad16, dtype=jnp.int32)
    row_p = jnp.concatenate([row, pad_i % N])
    col_p = jnp.concatenate([col, DUMMY + pad_i % (ACC_R - N)])
    rowx = jnp.stack([2 * row_p, 2 * row_p + 1]).reshape(NC, NS, ECH, K)
    col16 = col_p.reshape(NS, ECH, K)

    sc_hist, sc_scatter = _make_sc_kernels()
    hist = sc_hist(col3)
    hist_n = hist[:, :N, :1]

    h2 = pl.pallas_call(
        _tc_h2_body,
        out_shape=jax.ShapeDtypeStruct((N, C), jnp.float32),
    )(x, W, hist_n)
    h2v = h2.reshape(2 * N, HC)

    accs = sc_scatter(h2v, rowx, col16)

    out = pl.pallas_call(
        _tc_fin_body,
        out_shape=jax.ShapeDtypeStruct((N, C), jnp.float32),
    )(accs, h2, hist_n, b)
    return out
